# baseline (speedup 1.0000x reference)
import functools

import jax
import jax.numpy as jnp
from jax import lax
from jax.experimental import pallas as pl
from jax.experimental.pallas import tpu as pltpu
from jax.experimental.pallas import tpu_sc as plsc

_INTERPRET = False

T, D, E, K, F = 2048, 1024, 8, 2, 2048
B = 256
NB = (T * K) // B + E
G = T * K
GP = NB * B

NC, NS, L = 2, 16, 16
NW = NC * NS
CH = G // NW
NV = CH // L
RG = 32
NG = CH // RG


def _router_body(x_ref, wg_ref, w_ref, i_ref):
    logits = jnp.dot(x_ref[...], wg_ref[...], preferred_element_type=jnp.float32)
    tb = logits.shape[0]
    iota_e = lax.broadcasted_iota(jnp.int32, (tb, E), 1)
    m1 = jnp.max(logits, axis=1, keepdims=True)
    a1 = jnp.min(jnp.where(logits == m1, iota_e, E), axis=1, keepdims=True)
    masked = jnp.where(iota_e == a1, -jnp.inf, logits)
    m2 = jnp.max(masked, axis=1, keepdims=True)
    a2 = jnp.min(jnp.where(masked == m2, iota_e, E), axis=1, keepdims=True)
    w1 = jax.nn.sigmoid(m1 - m2)
    w_ref[...] = jnp.concatenate([w1, 1.0 - w1], axis=1)
    i_ref[...] = jnp.concatenate([a1, a2], axis=1).astype(jnp.int32)


def _router(x, wg):
    tb = 256
    return pl.pallas_call(
        _router_body,
        grid=(T // tb,),
        in_specs=[
            pl.BlockSpec((tb, D), lambda i: (i, 0)),
            pl.BlockSpec((D, E), lambda i: (0, 0)),
        ],
        out_specs=[
            pl.BlockSpec((tb, K), lambda i: (i, 0)),
            pl.BlockSpec((tb, K), lambda i: (i, 0)),
        ],
        out_shape=[
            jax.ShapeDtypeStruct((T, K), jnp.float32),
            jax.ShapeDtypeStruct((T, K), jnp.int32),
        ],
        interpret=_INTERPRET,
    )(x, wg)


def _dispatch_body(x_hbm, idx_hbm, xs_hbm, p_hbm, counts_hbm, bb_hbm,
                   idx_all, poff_ref, ppv, tokv, rows, cntv, bb_ref, sem):
    w = lax.axis_index("s") * NC + lax.axis_index("c")
    iota = lax.iota(jnp.int32, L)
    zero = jnp.zeros((L,), jnp.int32)

    pltpu.sync_copy(idx_hbm, idx_all)

    def hist(v, cnt):
        vec = idx_all[pl.ds(v * L, L)]
        for e in range(E):
            m = vec == e
            popc = plsc.all_reduce_population_count(m)
            cnt = cnt + jnp.where(iota == e, popc, zero)
        return cnt


    cnt_pre = lax.fori_loop(0, w * NV, hist, zero)
    cnt_full = lax.fori_loop(w * NV, G // L, hist, cnt_pre)

    nblk_b = ((cnt_full + (B - 1)) >> 8) << 8
    poff_ref[...] = plsc.cumsum(nblk_b) - nblk_b

    run = cnt_pre
    for j in range(NV):
        e_vec = idx_all[pl.ds((w * NV + j) * L, L)]
        rank = zero
        for e in range(E):
            m = e_vec == e
            c = plsc.cumsum(m.astype(jnp.int32))
            rank = jnp.where(m, run[e] + c - 1, rank)
            run = run + jnp.where(iota == e, plsc.all_reduce_population_count(m), zero)
        pp_vec = plsc.load_gather(poff_ref, [e_vec]) + rank
        tok_vec = ((w * CH + j * L) + iota) >> 1
        ppv[j // 2, pl.ds((j % 2) * L, L)] = pp_vec
        tokv[j // 2, pl.ds((j % 2) * L, L)] = tok_vec

    pltpu.sync_copy(ppv, p_hbm.at[w])

    @pl.when(w == 0)
    def _():
        nblk = (cnt_full + (B - 1)) >> 8
        blkc = plsc.cumsum(nblk)
        nreal = blkc[E - 1]
        be_last = jnp.int32(0)
        for e in range(E - 1):
            be_last = be_last + (blkc[e] <= nreal - 1).astype(jnp.int32)
        for half in range(2):
            bvec = iota + half * L
            be_raw = jnp.zeros((L,), jnp.int32)
            for e in range(E - 1):
                be_raw = be_raw + (blkc[e] <= bvec).astype(jnp.int32)
            val = (bvec < nreal).astype(jnp.int32)
            bev = jnp.where(val == 1, be_raw, be_last)
            cntv[...] = cnt_full
            bb_ref[pl.ds(half * L, L)] = bev
            bb_ref[pl.ds(2 * L + half * L, L)] = val
        pltpu.sync_copy(cntv, counts_hbm)
        pltpu.sync_copy(bb_ref, bb_hbm)

    cp = pltpu.async_copy(x_hbm.at[tokv.at[0]], rows.at[0], sem)
    for g in range(NG):
        cp.wait()
        if g + 1 < NG:
            cp = pltpu.async_copy(x_hbm.at[tokv.at[g + 1]], rows.at[(g + 1) % 2], sem)
        pltpu.sync_copy(rows.at[g % 2], xs_hbm.at[ppv.at[g]])


def _dispatch(x, idx_flat):
    mesh = plsc.VectorSubcoreMesh(core_axis_name="c", subcore_axis_name="s",
                                  num_cores=NC, num_subcores=NS)
    return pl.kernel(
        _dispatch_body,
        out_type=[
            jax.ShapeDtypeStruct((GP, D), jnp.float32),
            jax.ShapeDtypeStruct((NW, NG, RG), jnp.int32),
            jax.ShapeDtypeStruct((L,), jnp.int32),
            jax.ShapeDtypeStruct((4 * L,), jnp.int32),
        ],
        mesh=mesh,
        scratch_types=[
            pltpu.VMEM((G,), jnp.int32),
            pltpu.VMEM((L,), jnp.int32),
            pltpu.VMEM((NG, RG), jnp.int32),
            pltpu.VMEM((NG, RG), jnp.int32),
            pltpu.VMEM((2, RG, D), jnp.float32),
            pltpu.VMEM((L,), jnp.int32),
            pltpu.VMEM((4 * L,), jnp.int32),
            pltpu.SemaphoreType.DMA,
        ],
        compiler_params=pltpu.CompilerParams(needs_layout_passes=False),
    )(x, idx_flat)


def _wrap_dispatch(x, idx_flat):
    xs, p, counts, bb = _dispatch(x, idx_flat)
    return xs, p, counts, bb[:NB], bb[2 * L:2 * L + NB]


def _combine_body(out_hbm, p_hbm, wts_hbm, y_hbm, p_v, w_v, rows, ybuf, sem):
    w = lax.axis_index("s") * NC + lax.axis_index("c")
    pltpu.sync_copy(p_hbm.at[w], p_v)
    pltpu.sync_copy(wts_hbm.at[w], w_v)
    cp = pltpu.async_copy(out_hbm.at[p_v.at[0]], rows.at[0], sem)
    for g in range(NG):
        cp.wait()
        if g + 1 < NG:
            cp = pltpu.async_copy(out_hbm.at[p_v.at[g + 1]], rows.at[(g + 1) % 2], sem)
        rg = rows.at[g % 2]
        wv0 = w_v[g, pl.ds(0, L)]
        wv1 = w_v[g, pl.ds(L, L)]
        for i in range(RG // 2):
            wv = wv0 if 2 * i < L else wv1
            w0 = wv[(2 * i) % L]
            w1 = wv[(2 * i + 1) % L]

            def _mul(c):
                r0 = rg[2 * i, pl.ds(c, L)]
                r1 = rg[2 * i + 1, pl.ds(c, L)]
                ybuf[i, pl.ds(c, L)] = w0 * r0 + w1 * r1

            plsc.parallel_loop(0, D, L, unroll=8)(_mul)
        base = w * (T // NW) + g * (RG // 2)
        pltpu.sync_copy(ybuf, y_hbm.at[pl.ds(base, RG // 2)])


def _combine(out, p, wts):
    mesh = plsc.VectorSubcoreMesh(core_axis_name="c", subcore_axis_name="s",
                                  num_cores=NC, num_subcores=NS)
    return pl.kernel(
        _combine_body,
        out_type=jax.ShapeDtypeStruct((T, D), jnp.float32),
        mesh=mesh,
        scratch_types=[
            pltpu.VMEM((NG, RG), jnp.int32),
            pltpu.VMEM((NG, RG), jnp.float32),
            pltpu.VMEM((2, RG, D), jnp.float32),
            pltpu.VMEM((RG // 2, D), jnp.float32),
            pltpu.SemaphoreType.DMA,
        ],
    )(out, p, wts)


FS = 2
FT = F // FS


def _gmm_body(be_ref, valid_ref, xs_ref, w0_ref, w1_ref, wo_ref, out_ref,
              acc_ref):
    f, b = pl.program_id(0), pl.program_id(1)

    @pl.when(valid_ref[b] == 1)
    def _():
        xb = xs_ref[...].astype(jnp.bfloat16)
        h0 = jnp.dot(xb, w0_ref[0].astype(jnp.bfloat16),
                     preferred_element_type=jnp.float32)
        h1 = jnp.dot(xb, w1_ref[0].astype(jnp.bfloat16),
                     preferred_element_type=jnp.float32)
        act = (h0 * jax.nn.sigmoid(h0) * h1).astype(jnp.bfloat16)
        o = jnp.dot(act, wo_ref[0].astype(jnp.bfloat16),
                    preferred_element_type=jnp.float32)

        @pl.when(f == 0)
        def _():
            acc_ref[pl.ds(b * B, B), :] = o

        @pl.when(f == FS - 1)
        def _():
            out_ref[...] = acc_ref[pl.ds(b * B, B), :] + o


def _gmm(xs, w0, w1, wo, be, valid):
    grid_spec = pltpu.PrefetchScalarGridSpec(
        num_scalar_prefetch=2,
        grid=(FS, NB),
        in_specs=[
            pl.BlockSpec((B, D), lambda f, b, be, v: (b, 0)),
            pl.BlockSpec((1, D, FT), lambda f, b, be, v: (be[b], 0, f)),
            pl.BlockSpec((1, D, FT), lambda f, b, be, v: (be[b], 0, f)),
            pl.BlockSpec((1, FT, D), lambda f, b, be, v: (be[b], f, 0)),
        ],
        out_specs=pl.BlockSpec((B, D), lambda f, b, be, v: (b, 0)),
        scratch_shapes=[pltpu.VMEM((GP, D), jnp.float32)],
    )
    return pl.pallas_call(
        _gmm_body,
        grid_spec=grid_spec,
        out_shape=jax.ShapeDtypeStruct((GP, D), jnp.float32),
        interpret=_INTERPRET,
    )(be, valid, xs, w0, w1, wo)


def kernel(x, wg, w0, w1, wo):
    wts, idx = _router(x, wg)

    xs, p, counts, be, valid = _wrap_dispatch(x, idx.reshape(G))

    out = _gmm(xs, w0, w1, wo, be, valid)
    return _combine(out, p, wts.reshape(NW, NG, RG))


# speedup vs baseline: 1.1357x; 1.1357x over previous
import functools

import jax
import jax.numpy as jnp
from jax import lax
from jax.experimental import pallas as pl
from jax.experimental.pallas import tpu as pltpu
from jax.experimental.pallas import tpu_sc as plsc

_INTERPRET = False

T, D, E, K, F = 2048, 1024, 8, 2, 2048
B = 256
NB = (T * K) // B + E
G = T * K
GP = NB * B

NC, NS, L = 2, 16, 16
NW = NC * NS
CH = G // NW
NV = CH // L
RG = 32
NG = CH // RG


def _router_body(x_ref, wg_ref, w_ref, i_ref):
    logits = jnp.dot(x_ref[...], wg_ref[...], preferred_element_type=jnp.float32)
    tb = logits.shape[0]
    iota_e = lax.broadcasted_iota(jnp.int32, (tb, E), 1)
    m1 = jnp.max(logits, axis=1, keepdims=True)
    a1 = jnp.min(jnp.where(logits == m1, iota_e, E), axis=1, keepdims=True)
    masked = jnp.where(iota_e == a1, -jnp.inf, logits)
    m2 = jnp.max(masked, axis=1, keepdims=True)
    a2 = jnp.min(jnp.where(masked == m2, iota_e, E), axis=1, keepdims=True)
    w1 = jax.nn.sigmoid(m1 - m2)
    w_ref[...] = jnp.concatenate([w1, 1.0 - w1], axis=1)
    i_ref[...] = jnp.concatenate([a1, a2], axis=1).astype(jnp.int32)


def _router(x, wg):
    tb = 256
    return pl.pallas_call(
        _router_body,
        grid=(T // tb,),
        in_specs=[
            pl.BlockSpec((tb, D), lambda i: (i, 0)),
            pl.BlockSpec((D, E), lambda i: (0, 0)),
        ],
        out_specs=[
            pl.BlockSpec((tb, K), lambda i: (i, 0)),
            pl.BlockSpec((tb, K), lambda i: (i, 0)),
        ],
        out_shape=[
            jax.ShapeDtypeStruct((T, K), jnp.float32),
            jax.ShapeDtypeStruct((T, K), jnp.int32),
        ],
        interpret=_INTERPRET,
    )(x, wg)


def _dispatch_body(x_hbm, idx_hbm, xs_hbm, p_hbm, counts_hbm, bb_hbm,
                   idx_all, poff_ref, ppv, tokv, rows, cntv, bb_ref, sem):
    w = lax.axis_index("s") * NC + lax.axis_index("c")
    iota = lax.iota(jnp.int32, L)
    zero = jnp.zeros((L,), jnp.int32)

    pltpu.sync_copy(idx_hbm, idx_all)

    def hist(v, cnt):
        vec = idx_all[pl.ds(v * L, L)]
        for e in range(E):
            m = vec == e
            popc = plsc.all_reduce_population_count(m)
            cnt = cnt + jnp.where(iota == e, popc, zero)
        return cnt


    cnt_pre = lax.fori_loop(0, w * NV, hist, zero)
    cnt_full = lax.fori_loop(w * NV, G // L, hist, cnt_pre)

    nblk_b = ((cnt_full + (B - 1)) >> 8) << 8
    poff_ref[...] = plsc.cumsum(nblk_b) - nblk_b

    run = cnt_pre
    for j in range(NV):
        e_vec = idx_all[pl.ds((w * NV + j) * L, L)]
        rank = zero
        for e in range(E):
            m = e_vec == e
            c = plsc.cumsum(m.astype(jnp.int32))
            rank = jnp.where(m, run[e] + c - 1, rank)
            run = run + jnp.where(iota == e, plsc.all_reduce_population_count(m), zero)
        pp_vec = plsc.load_gather(poff_ref, [e_vec]) + rank
        tok_vec = ((w * CH + j * L) + iota) >> 1
        ppv[j // 2, pl.ds((j % 2) * L, L)] = pp_vec
        tokv[j // 2, pl.ds((j % 2) * L, L)] = tok_vec

    pltpu.sync_copy(ppv, p_hbm.at[w])

    @pl.when(w == 0)
    def _():
        nblk = (cnt_full + (B - 1)) >> 8
        blkc = plsc.cumsum(nblk)
        nreal = blkc[E - 1]
        be_last = jnp.int32(0)
        for e in range(E - 1):
            be_last = be_last + (blkc[e] <= nreal - 1).astype(jnp.int32)
        for half in range(2):
            bvec = iota + half * L
            be_raw = jnp.zeros((L,), jnp.int32)
            for e in range(E - 1):
                be_raw = be_raw + (blkc[e] <= bvec).astype(jnp.int32)
            val = (bvec < nreal).astype(jnp.int32)
            bev = jnp.where(val == 1, be_raw, be_last)
            cntv[...] = cnt_full
            bb_ref[pl.ds(half * L, L)] = bev
            bb_ref[pl.ds(2 * L + half * L, L)] = val
        pltpu.sync_copy(cntv, counts_hbm)
        pltpu.sync_copy(bb_ref, bb_hbm)

    cp = pltpu.async_copy(x_hbm.at[tokv.at[0]], rows.at[0], sem)
    for g in range(NG):
        cp.wait()
        if g + 1 < NG:
            cp = pltpu.async_copy(x_hbm.at[tokv.at[g + 1]], rows.at[(g + 1) % 2], sem)
        pltpu.sync_copy(rows.at[g % 2], xs_hbm.at[ppv.at[g]])


def _dispatch(x, idx_flat):
    mesh = plsc.VectorSubcoreMesh(core_axis_name="c", subcore_axis_name="s",
                                  num_cores=NC, num_subcores=NS)
    return pl.kernel(
        _dispatch_body,
        out_type=[
            jax.ShapeDtypeStruct((GP, D), jnp.float32),
            jax.ShapeDtypeStruct((NW, NG, RG), jnp.int32),
            jax.ShapeDtypeStruct((L,), jnp.int32),
            jax.ShapeDtypeStruct((4 * L,), jnp.int32),
        ],
        mesh=mesh,
        scratch_types=[
            pltpu.VMEM((G,), jnp.int32),
            pltpu.VMEM((L,), jnp.int32),
            pltpu.VMEM((NG, RG), jnp.int32),
            pltpu.VMEM((NG, RG), jnp.int32),
            pltpu.VMEM((2, RG, D), jnp.float32),
            pltpu.VMEM((L,), jnp.int32),
            pltpu.VMEM((4 * L,), jnp.int32),
            pltpu.SemaphoreType.DMA,
        ],
        compiler_params=pltpu.CompilerParams(needs_layout_passes=False),
    )(x, idx_flat)


def _wrap_dispatch(x, idx_flat):
    xs, p, counts, bb = _dispatch(x, idx_flat)
    return xs, p, counts, bb[:NB], bb[2 * L:2 * L + NB]


def _combine_body(out_hbm, p_hbm, wts_hbm, y_hbm, p_v, w_v, rows, ybuf, sem):
    w = lax.axis_index("s") * NC + lax.axis_index("c")
    pltpu.sync_copy(p_hbm.at[w], p_v)
    pltpu.sync_copy(wts_hbm.at[w], w_v)
    cp = pltpu.async_copy(out_hbm.at[p_v.at[0]], rows.at[0], sem)
    for g in range(NG):
        cp.wait()
        if g + 1 < NG:
            cp = pltpu.async_copy(out_hbm.at[p_v.at[g + 1]], rows.at[(g + 1) % 2], sem)
        rg = rows.at[g % 2]
        wv0 = w_v[g, pl.ds(0, L)]
        wv1 = w_v[g, pl.ds(L, L)]
        for i in range(RG // 2):
            wv = wv0 if 2 * i < L else wv1
            w0 = wv[(2 * i) % L]
            w1 = wv[(2 * i + 1) % L]

            def _mul(c):
                r0 = rg[2 * i, pl.ds(c, L)]
                r1 = rg[2 * i + 1, pl.ds(c, L)]
                ybuf[i, pl.ds(c, L)] = w0 * r0 + w1 * r1

            plsc.parallel_loop(0, D, L, unroll=8)(_mul)
        base = w * (T // NW) + g * (RG // 2)
        pltpu.sync_copy(ybuf, y_hbm.at[pl.ds(base, RG // 2)])


def _combine(out, p, wts):
    mesh = plsc.VectorSubcoreMesh(core_axis_name="c", subcore_axis_name="s",
                                  num_cores=NC, num_subcores=NS)
    return pl.kernel(
        _combine_body,
        out_type=jax.ShapeDtypeStruct((T, D), jnp.float32),
        mesh=mesh,
        scratch_types=[
            pltpu.VMEM((NG, RG), jnp.int32),
            pltpu.VMEM((NG, RG), jnp.float32),
            pltpu.VMEM((2, RG, D), jnp.float32),
            pltpu.VMEM((RG // 2, D), jnp.float32),
            pltpu.SemaphoreType.DMA,
        ],
    )(out, p, wts)


def _gmm_body(be_ref, valid_ref, xs_ref, w0_ref, w1_ref, wo_ref, out_ref):
    @pl.when(valid_ref[pl.program_id(0)] == 1)
    def _():
        xb = xs_ref[...].astype(jnp.bfloat16)
        h0 = jnp.dot(xb, w0_ref[0].astype(jnp.bfloat16),
                     preferred_element_type=jnp.float32)
        h1 = jnp.dot(xb, w1_ref[0].astype(jnp.bfloat16),
                     preferred_element_type=jnp.float32)
        act = (h0 * jax.nn.sigmoid(h0) * h1).astype(jnp.bfloat16)
        o = jnp.dot(act, wo_ref[0].astype(jnp.bfloat16),
                    preferred_element_type=jnp.float32)
        out_ref[...] = o


def _gmm(xs, w0, w1, wo, be, valid):
    grid_spec = pltpu.PrefetchScalarGridSpec(
        num_scalar_prefetch=2,
        grid=(NB,),
        in_specs=[
            pl.BlockSpec((B, D), lambda b, be, v: (b, 0)),
            pl.BlockSpec((1, D, F), lambda b, be, v: (be[b], 0, 0)),
            pl.BlockSpec((1, D, F), lambda b, be, v: (be[b], 0, 0)),
            pl.BlockSpec((1, F, D), lambda b, be, v: (be[b], 0, 0)),
        ],
        out_specs=pl.BlockSpec((B, D), lambda b, be, v: (b, 0)),
    )
    return pl.pallas_call(
        _gmm_body,
        grid_spec=grid_spec,
        out_shape=jax.ShapeDtypeStruct((GP, D), jnp.float32),
        compiler_params=pltpu.CompilerParams(
            vmem_limit_bytes=112 * 1024 * 1024),
        interpret=_INTERPRET,
    )(be, valid, xs, w0, w1, wo)


def kernel(x, wg, w0, w1, wo):
    wts, idx = _router(x, wg)

    xs, p, counts, be, valid = _wrap_dispatch(x, idx.reshape(G))

    out = _gmm(xs, w0, w1, wo, be, valid)
    return _combine(out, p, wts.reshape(NW, NG, RG))
